# baseline (speedup 1.0000x reference)
import jax
import jax.numpy as jnp
from jax import lax
from jax.experimental import pallas as pl
from jax.experimental.pallas import tpu as pltpu
from jax.experimental.pallas import tpu_sc as plsc

TABLE_SIZE = 262144
MASK = TABLE_SIZE - 1
PRIMES = (6700417, 15485863, 32452843, 49979687)
NUM_HEADS = 4
D = 128
L = 16
NC, NS = 2, 16
N_KEYS = 4096 * 64
KEYS_PER_W = N_KEYS // (NC * NS)
CHUNK = 64
N_CHUNKS = KEYS_PER_W // CHUNK
N_PAIRS = N_CHUNKS // 2


def _sc_body(keys_hbm, table_hbm, out_hbm, keys_v, idx_v, rows_v, out_v, sems):
    si = lax.axis_index("s")
    ci = lax.axis_index("c")
    gsem = [sems.at[jnp.int32(0)], sems.at[jnp.int32(1)]]
    wsem = [sems.at[jnp.int32(2)], sems.at[jnp.int32(3)]]

    pltpu.sync_copy(keys_hbm.at[si, ci], keys_v)

    def _hash_and_issue(g, buf):
        zvec = jnp.zeros((L,), jnp.float32)

        def _zero(i, carry):
            for cc in range(D // L):
                rows_v[buf, i, pl.ds(cc * L, L)] = zvec
            return carry

        lax.fori_loop(jnp.int32(0), jnp.int32(CHUNK), _zero, 0)
        off = g * jnp.int32(CHUNK)
        for v in range(CHUNK // L):
            k = keys_v[pl.ds(off + jnp.int32(v * L), L)]
            sl = pl.ds(v * L, L)
            for j in range(NUM_HEADS):
                idx_v[buf, j, sl] = (k * jnp.int32(PRIMES[j])) & jnp.int32(MASK)
        for j in range(NUM_HEADS):
            pltpu.async_copy(
                table_hbm.at[idx_v.at[jnp.int32(buf), jnp.int32(j)]],
                rows_v.at[jnp.int32(buf)],
                gsem[buf],
                add=True,
            )

    def _wait_gathers(buf):
        for j in range(NUM_HEADS):
            pltpu.make_async_copy(
                table_hbm.at[idx_v.at[jnp.int32(buf), jnp.int32(j)]],
                rows_v.at[jnp.int32(buf)],
                gsem[buf],
            ).wait()

    def _accumulate(buf):
        def _key(i, carry):
            for cc in range(D // L):
                sl = pl.ds(cc * L, L)
                out_v[buf, i, sl] = rows_v[buf, i, sl] * jnp.float32(0.25)
            return carry

        lax.fori_loop(jnp.int32(0), jnp.int32(CHUNK), _key, 0)

    def _wait_write(buf):
        pltpu.make_async_copy(
            out_v.at[jnp.int32(buf)], out_hbm.at[si, ci, jnp.int32(buf)], wsem[buf]
        ).wait()

    def _start_write(g, buf):
        pltpu.async_copy(
            out_v.at[jnp.int32(buf)], out_hbm.at[si, ci, g], wsem[buf]
        )

    _hash_and_issue(jnp.int32(0), 0)

    def _pair(c2, carry):
        g0 = c2 * jnp.int32(2)
        g1 = g0 + jnp.int32(1)

        _hash_and_issue(g1, 1)
        _wait_gathers(0)

        @pl.when(c2 > jnp.int32(0))
        def _():
            _wait_write(0)

        _accumulate(0)
        _start_write(g0, 0)

        @pl.when(c2 < jnp.int32(N_PAIRS - 1))
        def _():
            _hash_and_issue(g0 + jnp.int32(2), 0)

        _wait_gathers(1)

        @pl.when(c2 > jnp.int32(0))
        def _():
            _wait_write(1)

        _accumulate(1)
        _start_write(g1, 1)
        return carry

    lax.fori_loop(jnp.int32(0), jnp.int32(N_PAIRS), _pair, 0)
    _wait_write(0)
    _wait_write(1)


@jax.jit
def _sc_lookup(keys_grouped, table):
    mesh = plsc.VectorSubcoreMesh(
        core_axis_name="c", subcore_axis_name="s", num_cores=NC, num_subcores=NS
    )
    f = pl.kernel(
        _sc_body,
        out_type=jax.ShapeDtypeStruct((NS, NC, N_CHUNKS, CHUNK, D), jnp.float32),
        mesh=mesh,
        scratch_types=[
            pltpu.VMEM((KEYS_PER_W,), jnp.int32),
            pltpu.VMEM((2, NUM_HEADS, CHUNK), jnp.int32),
            pltpu.VMEM((2, CHUNK, D), jnp.float32),
            pltpu.VMEM((2, CHUNK, D), jnp.float32),
            pltpu.SemaphoreType.DMA((4,)),
        ],
    )
    return f(keys_grouped, table)


def kernel(keys, table):
    M, G = keys.shape
    keys_grouped = keys.reshape(NS, NC, KEYS_PER_W).astype(jnp.int32)
    out = _sc_lookup(keys_grouped, table)
    return out.reshape(M, G, table.shape[1])


# speedup vs baseline: 1.0188x; 1.0188x over previous
import jax
import jax.numpy as jnp
from jax import lax
from jax.experimental import pallas as pl
from jax.experimental.pallas import tpu as pltpu
from jax.experimental.pallas import tpu_sc as plsc

TABLE_SIZE = 262144
MASK = TABLE_SIZE - 1
PRIMES = (6700417, 15485863, 32452843, 49979687)
NUM_HEADS = 4
D = 128
L = 16
NC, NS = 2, 16
N_KEYS = 4096 * 64
KEYS_PER_W = N_KEYS // (NC * NS)
CHUNK = 128
N_CHUNKS = KEYS_PER_W // CHUNK
N_PAIRS = N_CHUNKS // 2


def _sc_body(keys_hbm, table_hbm, out_hbm, keys_v, idx_v, rows_v, out_v, sems):
    si = lax.axis_index("s")
    ci = lax.axis_index("c")
    gsem = [sems.at[jnp.int32(0)], sems.at[jnp.int32(1)]]
    wsem = [sems.at[jnp.int32(2)], sems.at[jnp.int32(3)]]

    pltpu.sync_copy(keys_hbm.at[si, ci], keys_v)

    zvec = jnp.zeros((L,), jnp.float32)

    def _zero(buf):
        def _z(i, carry):
            for cc in range(D // L):
                rows_v[buf, i, pl.ds(cc * L, L)] = zvec
            return carry

        lax.fori_loop(jnp.int32(0), jnp.int32(CHUNK), _z, 0)

    def _hash_and_issue(g, buf):
        off = g * jnp.int32(CHUNK)
        for v in range(CHUNK // L):
            k = keys_v[pl.ds(off + jnp.int32(v * L), L)]
            sl = pl.ds(v * L, L)
            for j in range(NUM_HEADS):
                idx_v[buf, j, sl] = (k * jnp.int32(PRIMES[j])) & jnp.int32(MASK)
        for j in range(NUM_HEADS):
            pltpu.async_copy(
                table_hbm.at[idx_v.at[jnp.int32(buf), jnp.int32(j)]],
                rows_v.at[jnp.int32(buf)],
                gsem[buf],
                add=True,
            )

    def _wait_gathers(buf):
        for j in range(NUM_HEADS):
            pltpu.make_async_copy(
                table_hbm.at[idx_v.at[jnp.int32(buf), jnp.int32(j)]],
                rows_v.at[jnp.int32(buf)],
                gsem[buf],
            ).wait()

    def _accumulate(buf):
        def _key(i, carry):
            for cc in range(D // L):
                sl = pl.ds(cc * L, L)
                out_v[buf, i, sl] = rows_v[buf, i, sl] * jnp.float32(0.25)
                rows_v[buf, i, sl] = zvec
            return carry

        lax.fori_loop(jnp.int32(0), jnp.int32(CHUNK), _key, 0)

    def _wait_write(buf):
        pltpu.make_async_copy(
            out_v.at[jnp.int32(buf)], out_hbm.at[si, ci, jnp.int32(buf)], wsem[buf]
        ).wait()

    def _start_write(g, buf):
        pltpu.async_copy(
            out_v.at[jnp.int32(buf)], out_hbm.at[si, ci, g], wsem[buf]
        )

    _zero(0)
    _zero(1)
    _hash_and_issue(jnp.int32(0), 0)

    def _pair(c2, carry):
        g0 = c2 * jnp.int32(2)
        g1 = g0 + jnp.int32(1)

        _hash_and_issue(g1, 1)
        _wait_gathers(0)

        @pl.when(c2 > jnp.int32(0))
        def _():
            _wait_write(0)

        _accumulate(0)
        _start_write(g0, 0)

        @pl.when(c2 < jnp.int32(N_PAIRS - 1))
        def _():
            _hash_and_issue(g0 + jnp.int32(2), 0)

        _wait_gathers(1)

        @pl.when(c2 > jnp.int32(0))
        def _():
            _wait_write(1)

        _accumulate(1)
        _start_write(g1, 1)
        return carry

    lax.fori_loop(jnp.int32(0), jnp.int32(N_PAIRS), _pair, 0)
    _wait_write(0)
    _wait_write(1)


@jax.jit
def _sc_lookup(keys_grouped, table):
    mesh = plsc.VectorSubcoreMesh(
        core_axis_name="c", subcore_axis_name="s", num_cores=NC, num_subcores=NS
    )
    f = pl.kernel(
        _sc_body,
        out_type=jax.ShapeDtypeStruct((NS, NC, N_CHUNKS, CHUNK, D), jnp.float32),
        mesh=mesh,
        scratch_types=[
            pltpu.VMEM((KEYS_PER_W,), jnp.int32),
            pltpu.VMEM((2, NUM_HEADS, CHUNK), jnp.int32),
            pltpu.VMEM((2, CHUNK, D), jnp.float32),
            pltpu.VMEM((2, CHUNK, D), jnp.float32),
            pltpu.SemaphoreType.DMA((4,)),
        ],
    )
    return f(keys_grouped, table)


def kernel(keys, table):
    M, G = keys.shape
    keys_grouped = keys.reshape(NS, NC, KEYS_PER_W).astype(jnp.int32)
    out = _sc_lookup(keys_grouped, table)
    return out.reshape(M, G, table.shape[1])


# speedup vs baseline: 1.0236x; 1.0048x over previous
import jax
import jax.numpy as jnp
from jax import lax
from jax.experimental import pallas as pl
from jax.experimental.pallas import tpu as pltpu
from jax.experimental.pallas import tpu_sc as plsc

TABLE_SIZE = 262144
MASK = TABLE_SIZE - 1
PRIMES = (6700417, 15485863, 32452843, 49979687)
NUM_HEADS = 4
D = 128
L = 16
NC, NS = 2, 16
N_KEYS = 4096 * 64
KEYS_PER_W = N_KEYS // (NC * NS)
IDXW = 128
KC = 2
CHUNK = KC * IDXW
N_CHUNKS = KEYS_PER_W // CHUNK
N_PAIRS = N_CHUNKS // 2


def _sc_body(keys_hbm, table_hbm, out_hbm, keys_v, idx_v, rows_v, sems):
    si = lax.axis_index("s")
    ci = lax.axis_index("c")
    gsem = [sems.at[jnp.int32(0)], sems.at[jnp.int32(1)]]
    wsem = [sems.at[jnp.int32(2)], sems.at[jnp.int32(3)]]

    pltpu.sync_copy(keys_hbm.at[si, ci], keys_v)

    zvec = jnp.zeros((L,), jnp.float32)

    def _zero(buf):
        def _z(i, carry):
            for r in range(KC):
                for cc in range(D // L):
                    rows_v[buf, r, i, pl.ds(cc * L, L)] = zvec
            return carry

        lax.fori_loop(jnp.int32(0), jnp.int32(IDXW), _z, 0)

    def _hash_and_issue(g, buf):
        off = g * jnp.int32(CHUNK)
        for v in range(CHUNK // L):
            k = keys_v[pl.ds(off + jnp.int32(v * L), L)]
            r, col = divmod(v * L, IDXW)
            sl = pl.ds(col, L)
            for j in range(NUM_HEADS):
                idx_v[buf, j, r, sl] = (k * jnp.int32(PRIMES[j])) & jnp.int32(MASK)
        for j in range(NUM_HEADS):
            for r in range(KC):
                pltpu.async_copy(
                    table_hbm.at[idx_v.at[jnp.int32(buf), jnp.int32(j), jnp.int32(r)]],
                    rows_v.at[jnp.int32(buf), jnp.int32(r)],
                    gsem[buf],
                    add=True,
                )

    def _wait_gathers(buf):
        for j in range(NUM_HEADS):
            for r in range(KC):
                pltpu.make_async_copy(
                    table_hbm.at[idx_v.at[jnp.int32(buf), jnp.int32(j), jnp.int32(r)]],
                    rows_v.at[jnp.int32(buf), jnp.int32(r)],
                    gsem[buf],
                ).wait()

    def _scale(buf):
        def _key(i, carry):
            for r in range(KC):
                for cc in range(D // L):
                    sl = pl.ds(cc * L, L)
                    rows_v[buf, r, i, sl] = rows_v[buf, r, i, sl] * jnp.float32(0.25)
            return carry

        lax.fori_loop(jnp.int32(0), jnp.int32(IDXW), _key, 0)

    def _wait_write(buf):
        pltpu.make_async_copy(
            rows_v.at[jnp.int32(buf)], out_hbm.at[si, ci, jnp.int32(buf)], wsem[buf]
        ).wait()

    def _start_write(g, buf):
        pltpu.async_copy(
            rows_v.at[jnp.int32(buf)], out_hbm.at[si, ci, g], wsem[buf]
        )

    _zero(0)
    _zero(1)
    _hash_and_issue(jnp.int32(0), 0)
    _hash_and_issue(jnp.int32(1), 1)

    def _pair(c2, carry):
        g0 = c2 * jnp.int32(2)
        g1 = g0 + jnp.int32(1)

        _wait_gathers(0)
        _scale(0)
        _start_write(g0, 0)

        @pl.when(c2 < jnp.int32(N_PAIRS - 1))
        def _():
            _wait_write(0)
            _zero(0)
            _hash_and_issue(g0 + jnp.int32(2), 0)

        _wait_gathers(1)
        _scale(1)
        _start_write(g1, 1)

        @pl.when(c2 < jnp.int32(N_PAIRS - 1))
        def _():
            _wait_write(1)
            _zero(1)
            _hash_and_issue(g0 + jnp.int32(3), 1)

        return carry

    lax.fori_loop(jnp.int32(0), jnp.int32(N_PAIRS), _pair, 0)
    _wait_write(0)
    _wait_write(1)


@jax.jit
def _sc_lookup(keys_grouped, table):
    mesh = plsc.VectorSubcoreMesh(
        core_axis_name="c", subcore_axis_name="s", num_cores=NC, num_subcores=NS
    )
    f = pl.kernel(
        _sc_body,
        out_type=jax.ShapeDtypeStruct((NS, NC, N_CHUNKS, KC, IDXW, D), jnp.float32),
        mesh=mesh,
        scratch_types=[
            pltpu.VMEM((KEYS_PER_W,), jnp.int32),
            pltpu.VMEM((2, NUM_HEADS, KC, IDXW), jnp.int32),
            pltpu.VMEM((2, KC, IDXW, D), jnp.float32),
            pltpu.SemaphoreType.DMA((4,)),
        ],
    )
    return f(keys_grouped, table)


def kernel(keys, table):
    M, G = keys.shape
    keys_grouped = keys.reshape(NS, NC, KEYS_PER_W).astype(jnp.int32)
    out = _sc_lookup(keys_grouped, table)
    return out.reshape(M, G, table.shape[1])
